# Initial kernel scaffold; baseline (speedup 1.0000x reference)
import jax
import jax.numpy as jnp
from jax.experimental import pallas as pl


def kernel(x, edge_index, W1, att_src1, att_dst1, b1, W2, att_src2, att_dst2, b2, fc1_w, fc1_b, fc2_w, fc2_b):
    raise NotImplementedError("write your pallas kernel here")


# speedup vs baseline: 26.4732x; 26.4732x over previous
import functools

import jax
import jax.numpy as jnp
from jax import lax
from jax.experimental import pallas as pl
from jax.experimental.pallas import tpu as pltpu
from jax.experimental.pallas import tpu_sc as plsc

NN = 10000
EE = 320000
NFEAT = 128
NHID = 8
HEADS = 8
NCLASS = 16
LBDA0 = 0.5
LBDA1 = 0.3

NC, NS = 2, 16
NW = NC * NS
CHUNK = 64
NPAD = 10240
ROWS_PER_TILE = NPAD // NS
ETOT = EE + NN
EPW = 10368
EP = EPW * NW
NCHUNKS = EPW // CHUNK
ZROWS = 64

AROW1 = 128
ROW1, BROW1, VAL1 = 80, 16, 64
ROW2, BROW2, VAL2 = 32, 16, 16


def _make_sc_edge(arow_w, row_w, brow_w, val_w, n_heads, head_w, stage_a):
  mesh = plsc.VectorSubcoreMesh(core_axis_name="c", subcore_axis_name="s")

  def body(src_hbm, dst_hbm, taba_hbm, tabb_hbm, out_hbm,
           srcb, dstb, rowsa, rowsb, scaled, zbuf, tabs_a, tabs_b, acc,
           sem_a, sem_b, sem_s):
    c = lax.axis_index("c")
    s = lax.axis_index("s")
    wid = s * NC + c

    zv = jnp.zeros((16,), jnp.float32)

    def zb_body(i, _):
      for j in range(row_w // 16):
        zbuf[i, pl.ds(j * 16, 16)] = zv
      return 0

    def zs_body(i, _):
      for j in range(row_w // 16):
        scaled[i, pl.ds(j * 16, 16)] = zv
      return 0

    lax.fori_loop(0, ZROWS, zb_body, 0)
    lax.fori_loop(0, CHUNK, zs_body, 0)
    trow = s * ROWS_PER_TILE
    if stage_a:
      pltpu.sync_copy(taba_hbm.at[pl.ds(trow, ROWS_PER_TILE)],
                      tabs_a.at[pl.ds(trow, ROWS_PER_TILE)])
    pltpu.sync_copy(tabb_hbm.at[pl.ds(trow, ROWS_PER_TILE)],
                    tabs_b.at[pl.ds(trow, ROWS_PER_TILE)])
    for i in range(ROWS_PER_TILE // ZROWS):
      pltpu.sync_copy(zbuf, acc.at[pl.ds(trow + i * ZROWS, ZROWS)])
    plsc.subcore_barrier()

    iota16 = lax.iota(jnp.int32, 16)

    def group_body(g, _):
      eids = iota16 + g * 16
      exs = []
      for h in range(n_heads):
        av = plsc.load_gather(rowsa, [eids, jnp.full((16,), val_w + h, jnp.int32)])
        bv = plsc.load_gather(rowsb, [eids, jnp.full((16,), h, jnp.int32)])
        v = av + bv
        e = jnp.exp(jnp.maximum(v, 0.2 * v))
        exs.append(e)
        plsc.store_scatter(scaled, [eids, jnp.full((16,), val_w + h, jnp.int32)], e)
      for cc in range(val_w):
        col = plsc.load_gather(rowsa, [eids, jnp.full((16,), cc, jnp.int32)])
        plsc.store_scatter(scaled, [eids, jnp.full((16,), cc, jnp.int32)],
                           col * exs[cc // head_w])
      return 0

    a_src = tabs_a if stage_a else taba_hbm

    def chunk_body(i, _):
      base = wid * EPW + i * CHUNK
      pltpu.sync_copy(src_hbm.at[pl.ds(base, CHUNK)], srcb)
      pltpu.sync_copy(dst_hbm.at[pl.ds(base, CHUNK)], dstb)
      ga = pltpu.async_copy(a_src.at[srcb], rowsa, sem_a)
      gb = pltpu.async_copy(tabs_b.at[dstb], rowsb, sem_b)
      ga.wait()
      gb.wait()
      lax.fori_loop(0, CHUNK // 16, group_body, 0)
      pltpu.async_copy(scaled, acc.at[dstb], sem_s, add=True).wait()
      return 0

    lax.fori_loop(0, NCHUNKS, chunk_body, 0)
    plsc.subcore_barrier()
    pltpu.sync_copy(acc.at[pl.ds(trow, ROWS_PER_TILE)],
                    out_hbm.at[c, pl.ds(trow, ROWS_PER_TILE)])

  scratch = [
      pltpu.VMEM((CHUNK,), jnp.int32),
      pltpu.VMEM((CHUNK,), jnp.int32),
      pltpu.VMEM((CHUNK, arow_w), jnp.float32),
      pltpu.VMEM((CHUNK, brow_w), jnp.float32),
      pltpu.VMEM((CHUNK, row_w), jnp.float32),
      pltpu.VMEM((ZROWS, row_w), jnp.float32),
      (pltpu.VMEM_SHARED((NPAD, arow_w), jnp.float32) if stage_a
       else pltpu.VMEM_SHARED((8, 8), jnp.float32)),
      pltpu.VMEM_SHARED((NPAD, brow_w), jnp.float32),
      pltpu.VMEM_SHARED((NPAD, row_w), jnp.float32),
      pltpu.SemaphoreType.DMA,
      pltpu.SemaphoreType.DMA,
      pltpu.SemaphoreType.DMA,
  ]
  return pl.kernel(
      body,
      out_type=jax.ShapeDtypeStruct((NC, NPAD, row_w), jnp.float32),
      mesh=mesh,
      compiler_params=pltpu.CompilerParams(needs_layout_passes=False),
      scratch_types=scratch,
  )


def _tc1_body(x_ref, w1_ref, bds_ref, bdd_ref, taba_ref, tabb_ref):
  h1 = jnp.dot(x_ref[...], w1_ref[...], preferred_element_type=jnp.float32)
  asrc = jnp.dot(h1, bds_ref[...], preferred_element_type=jnp.float32)
  adst = jnp.dot(h1, bdd_ref[...], preferred_element_type=jnp.float32)
  z8 = jnp.zeros_like(asrc)
  zpad = jnp.zeros((h1.shape[0], AROW1 - VAL1 - HEADS), jnp.float32)
  taba_ref[...] = jnp.concatenate([h1, asrc, zpad], axis=1)
  tabb_ref[...] = jnp.concatenate([adst, z8], axis=1)


def _tc2_body(acc_ref, x_ref, fc1_ref, cvec_ref, rep8_ref, w2_ref, att2_ref,
              fc2_ref, taba2_ref, tabb2_ref, side2_ref):
  num = acc_ref[0, :, 0:VAL1] + acc_ref[1, :, 0:VAL1]
  den = acc_ref[0, :, VAL1:VAL1 + HEADS] + acc_ref[1, :, VAL1:VAL1 + HEADS]
  denx = jnp.dot(den, rep8_ref[...], preferred_element_type=jnp.float32)
  out1 = num / (denx + 1e-16)
  side1 = jnp.dot(x_ref[...], fc1_ref[...], preferred_element_type=jnp.float32)
  v = out1 + cvec_ref[0:1, :] - LBDA0 * side1
  h = jnp.where(v > 0, v, jnp.exp(jnp.minimum(v, 0.0)) - 1.0)
  h2 = jnp.dot(h, w2_ref[...], preferred_element_type=jnp.float32)
  asrc2 = jnp.sum(h2 * att2_ref[0:1, :], axis=1, keepdims=True)
  adst2 = jnp.sum(h2 * att2_ref[1:2, :], axis=1, keepdims=True)
  zpad2 = jnp.zeros((h2.shape[0], AROW1 - VAL2 - 1), jnp.float32)
  z15 = jnp.zeros((h2.shape[0], BROW2 - 1), jnp.float32)
  taba2_ref[...] = jnp.concatenate([h2, asrc2, zpad2], axis=1)
  tabb2_ref[...] = jnp.concatenate([adst2, z15], axis=1)
  side2_ref[...] = jnp.dot(h, fc2_ref[...], preferred_element_type=jnp.float32)


def _tc3_body(acc_ref, side_ref, c2_ref, out_ref):
  num = acc_ref[0, :, 0:VAL2] + acc_ref[1, :, 0:VAL2]
  den = acc_ref[0, :, VAL2:VAL2 + 1] + acc_ref[1, :, VAL2:VAL2 + 1]
  z = num / (den + 1e-16) + c2_ref[0:1, :] - LBDA1 * side_ref[...]
  m = jnp.max(z, axis=1, keepdims=True)
  zs = z - m
  lse = jnp.log(jnp.sum(jnp.exp(zs), axis=1, keepdims=True))
  out_ref[...] = zs - lse


def kernel(x, edge_index, W1, att_src1, att_dst1, b1, W2, att_src2, att_dst2,
           b2, fc1_w, fc1_b, fc2_w, fc2_b):
  f32 = jnp.float32
  xp = jnp.pad(x, ((0, NPAD - NN), (0, 0)))
  eye8 = jnp.eye(8, dtype=f32)
  bds = (eye8[:, None, :] * att_src1.reshape(HEADS, NHID)[:, :, None]).reshape(HEADS * NHID, HEADS)
  bdd = (eye8[:, None, :] * att_dst1.reshape(HEADS, NHID)[:, :, None]).reshape(HEADS * NHID, HEADS)
  rep8 = jnp.repeat(eye8, NHID, axis=1)
  fc1t = jnp.tile(fc1_w.T, (1, HEADS))
  cvec = jnp.broadcast_to(b1 - LBDA0 * jnp.tile(fc1_b, HEADS), (8, HEADS * NHID))
  att2 = jnp.concatenate([att_src2.reshape(1, NCLASS), att_dst2.reshape(1, NCLASS),
                          jnp.zeros((6, NCLASS), f32)], axis=0)
  fc2t = fc2_w.T
  c2vec = jnp.broadcast_to(b2 - LBDA1 * fc2_b, (8, NCLASS))

  ar = jnp.arange(NN, dtype=jnp.int32)
  padidx = NN + (jnp.arange(EP - ETOT, dtype=jnp.int32) % 64)
  src = jnp.concatenate([edge_index[0], ar, padidx])
  dst = jnp.concatenate([edge_index[1], ar, padidx])

  tab_a, tab_b = pl.pallas_call(
      _tc1_body,
      grid=(NPAD // 256,),
      in_specs=[
          pl.BlockSpec((256, NFEAT), lambda i: (i, 0)),
          pl.BlockSpec((NFEAT, HEADS * NHID), lambda i: (0, 0)),
          pl.BlockSpec((HEADS * NHID, HEADS), lambda i: (0, 0)),
          pl.BlockSpec((HEADS * NHID, HEADS), lambda i: (0, 0)),
      ],
      out_specs=[
          pl.BlockSpec((256, AROW1), lambda i: (i, 0)),
          pl.BlockSpec((256, BROW1), lambda i: (i, 0)),
      ],
      out_shape=[
          jax.ShapeDtypeStruct((NPAD, AROW1), f32),
          jax.ShapeDtypeStruct((NPAD, BROW1), f32),
      ],
  )(xp, W1, bds, bdd)

  acc1 = _make_sc_edge(AROW1, ROW1, BROW1, VAL1, HEADS, NHID,
                       stage_a=False)(src, dst, tab_a, tab_b)

  taba2, tabb2, side2 = pl.pallas_call(
      _tc2_body,
      grid=(NPAD // 256,),
      in_specs=[
          pl.BlockSpec((2, 256, ROW1), lambda i: (0, i, 0)),
          pl.BlockSpec((256, NFEAT), lambda i: (i, 0)),
          pl.BlockSpec((NFEAT, HEADS * NHID), lambda i: (0, 0)),
          pl.BlockSpec((8, HEADS * NHID), lambda i: (0, 0)),
          pl.BlockSpec((HEADS, HEADS * NHID), lambda i: (0, 0)),
          pl.BlockSpec((HEADS * NHID, NCLASS), lambda i: (0, 0)),
          pl.BlockSpec((8, NCLASS), lambda i: (0, 0)),
          pl.BlockSpec((HEADS * NHID, NCLASS), lambda i: (0, 0)),
      ],
      out_specs=[
          pl.BlockSpec((256, AROW1), lambda i: (i, 0)),
          pl.BlockSpec((256, BROW2), lambda i: (i, 0)),
          pl.BlockSpec((256, NCLASS), lambda i: (i, 0)),
      ],
      out_shape=[
          jax.ShapeDtypeStruct((NPAD, AROW1), f32),
          jax.ShapeDtypeStruct((NPAD, BROW2), f32),
          jax.ShapeDtypeStruct((NPAD, NCLASS), f32),
      ],
  )(acc1, xp, fc1t, cvec, rep8, W2, att2, fc2t)

  acc2 = _make_sc_edge(AROW1, ROW2, BROW2, VAL2, 1, NCLASS,
                       stage_a=False)(src, dst, taba2, tabb2)

  out = pl.pallas_call(
      _tc3_body,
      grid=(NN // 400,),
      in_specs=[
          pl.BlockSpec((2, 400, ROW2), lambda i: (0, i, 0)),
          pl.BlockSpec((400, NCLASS), lambda i: (i, 0)),
          pl.BlockSpec((8, NCLASS), lambda i: (0, 0)),
      ],
      out_specs=pl.BlockSpec((400, NCLASS), lambda i: (i, 0)),
      out_shape=jax.ShapeDtypeStruct((NN, NCLASS), f32),
  )(acc2, side2, c2vec)

  return out


# speedup vs baseline: 26.4749x; 1.0001x over previous
import jax
import jax.numpy as jnp
from jax import lax
from jax.experimental import pallas as pl
from jax.experimental.pallas import tpu as pltpu
from jax.experimental.pallas import tpu_sc as plsc

NN = 10000
EE = 320000
NFEAT = 128
NHID = 8
HEADS = 8
NCLASS = 16
LBDA0 = 0.5
LBDA1 = 0.3

NC, NS = 2, 16
NW = NC * NS
CHUNK = 64
NPAD = 10240
ROWS_PER_TILE = NPAD // NS
ETOT = EE + NN
EPW = 10368
EP = EPW * NW
NCHUNKS = EPW // CHUNK
ZROWS = 64

AROW1 = 128
ROW1, BROW1, VAL1 = 80, 16, 64
ROW2, BROW2, VAL2 = 32, 16, 16


def _make_sc_edge(arow_w, row_w, brow_w, val_w, n_heads, head_w, stage_a):
  mesh = plsc.VectorSubcoreMesh(core_axis_name="c", subcore_axis_name="s")

  def body(src_hbm, dst_hbm, taba_hbm, tabb_hbm, out_hbm,
           srcb, dstb, rowsa, rowsb, scaled, zbuf, tabs_a, tabs_b, acc,
           sem_a, sem_b, sem_s):
    c = lax.axis_index("c")
    s = lax.axis_index("s")
    wid = s * NC + c

    zv = jnp.zeros((16,), jnp.float32)

    def zb_body(i, _):
      for j in range(row_w // 16):
        zbuf[i, pl.ds(j * 16, 16)] = zv
      return 0

    def zs_body(i, _):
      for j in range(row_w // 16):
        scaled[i, pl.ds(j * 16, 16)] = zv
      return 0

    lax.fori_loop(0, ZROWS, zb_body, 0)
    lax.fori_loop(0, CHUNK, zs_body, 0)
    trow = s * ROWS_PER_TILE
    if stage_a:
      pltpu.sync_copy(taba_hbm.at[pl.ds(trow, ROWS_PER_TILE)],
                      tabs_a.at[pl.ds(trow, ROWS_PER_TILE)])
    pltpu.sync_copy(tabb_hbm.at[pl.ds(trow, ROWS_PER_TILE)],
                    tabs_b.at[pl.ds(trow, ROWS_PER_TILE)])
    for i in range(ROWS_PER_TILE // ZROWS):
      pltpu.sync_copy(zbuf, acc.at[pl.ds(trow + i * ZROWS, ZROWS)])
    plsc.subcore_barrier()

    iota16 = lax.iota(jnp.int32, 16)

    def group_body(g, _):
      eids = iota16 + g * 16
      exs = []
      for h in range(n_heads):
        av = plsc.load_gather(rowsa, [eids, jnp.full((16,), val_w + h, jnp.int32)])
        bv = plsc.load_gather(rowsb, [eids, jnp.full((16,), h, jnp.int32)])
        v = av + bv
        e = jnp.exp(jnp.maximum(v, 0.2 * v))
        exs.append(e)
        plsc.store_scatter(scaled, [eids, jnp.full((16,), val_w + h, jnp.int32)], e)
      for cc in range(val_w):
        col = plsc.load_gather(rowsa, [eids, jnp.full((16,), cc, jnp.int32)])
        plsc.store_scatter(scaled, [eids, jnp.full((16,), cc, jnp.int32)],
                           col * exs[cc // head_w])
      return 0

    a_src = tabs_a if stage_a else taba_hbm

    def chunk_body(i, _):
      base = wid * EPW + i * CHUNK
      pltpu.sync_copy(src_hbm.at[pl.ds(base, CHUNK)], srcb)
      pltpu.sync_copy(dst_hbm.at[pl.ds(base, CHUNK)], dstb)
      ga = pltpu.async_copy(a_src.at[srcb], rowsa, sem_a)
      gb = pltpu.async_copy(tabs_b.at[dstb], rowsb, sem_b)
      ga.wait()
      gb.wait()
      lax.fori_loop(0, CHUNK // 16, group_body, 0)
      pltpu.async_copy(scaled, acc.at[dstb], sem_s, add=True).wait()
      return 0

    lax.fori_loop(0, NCHUNKS, chunk_body, 0)
    plsc.subcore_barrier()
    pltpu.sync_copy(acc.at[pl.ds(trow, ROWS_PER_TILE)],
                    out_hbm.at[c, pl.ds(trow, ROWS_PER_TILE)])

  scratch = [
      pltpu.VMEM((CHUNK,), jnp.int32),
      pltpu.VMEM((CHUNK,), jnp.int32),
      pltpu.VMEM((CHUNK, arow_w), jnp.float32),
      pltpu.VMEM((CHUNK, brow_w), jnp.float32),
      pltpu.VMEM((CHUNK, row_w), jnp.float32),
      pltpu.VMEM((ZROWS, row_w), jnp.float32),
      (pltpu.VMEM_SHARED((NPAD, arow_w), jnp.float32) if stage_a
       else pltpu.VMEM_SHARED((8, 8), jnp.float32)),
      pltpu.VMEM_SHARED((NPAD, brow_w), jnp.float32),
      pltpu.VMEM_SHARED((NPAD, row_w), jnp.float32),
      pltpu.SemaphoreType.DMA,
      pltpu.SemaphoreType.DMA,
      pltpu.SemaphoreType.DMA,
  ]
  return pl.kernel(
      body,
      out_type=jax.ShapeDtypeStruct((NC, NPAD, row_w), jnp.float32),
      mesh=mesh,
      compiler_params=pltpu.CompilerParams(needs_layout_passes=False),
      scratch_types=scratch,
  )


def _tc1_body(x_ref, w1_ref, bds_ref, bdd_ref, taba_ref, tabb_ref):
  h1 = jnp.dot(x_ref[...], w1_ref[...], preferred_element_type=jnp.float32)
  asrc = jnp.dot(h1, bds_ref[...], preferred_element_type=jnp.float32)
  adst = jnp.dot(h1, bdd_ref[...], preferred_element_type=jnp.float32)
  z8 = jnp.zeros_like(asrc)
  zpad = jnp.zeros((h1.shape[0], AROW1 - VAL1 - HEADS), jnp.float32)
  taba_ref[...] = jnp.concatenate([h1, asrc, zpad], axis=1)
  tabb_ref[...] = jnp.concatenate([adst, z8], axis=1)


def _tc2_body(acc_ref, x_ref, fc1_ref, cvec_ref, rep8_ref, w2_ref, att2_ref,
              fc2_ref, taba2_ref, tabb2_ref, side2_ref):
  num = acc_ref[0, :, 0:VAL1] + acc_ref[1, :, 0:VAL1]
  den = acc_ref[0, :, VAL1:VAL1 + HEADS] + acc_ref[1, :, VAL1:VAL1 + HEADS]
  denx = jnp.dot(den, rep8_ref[...], preferred_element_type=jnp.float32)
  out1 = num / (denx + 1e-16)
  side1 = jnp.dot(x_ref[...], fc1_ref[...], preferred_element_type=jnp.float32)
  v = out1 + cvec_ref[0:1, :] - LBDA0 * side1
  h = jnp.where(v > 0, v, jnp.exp(jnp.minimum(v, 0.0)) - 1.0)
  h2 = jnp.dot(h, w2_ref[...], preferred_element_type=jnp.float32)
  asrc2 = jnp.sum(h2 * att2_ref[0:1, :], axis=1, keepdims=True)
  adst2 = jnp.sum(h2 * att2_ref[1:2, :], axis=1, keepdims=True)
  zpad2 = jnp.zeros((h2.shape[0], AROW1 - VAL2 - 1), jnp.float32)
  z15 = jnp.zeros((h2.shape[0], BROW2 - 1), jnp.float32)
  taba2_ref[...] = jnp.concatenate([h2, asrc2, zpad2], axis=1)
  tabb2_ref[...] = jnp.concatenate([adst2, z15], axis=1)
  side2_ref[...] = jnp.dot(h, fc2_ref[...], preferred_element_type=jnp.float32)


def _tc3_body(acc_ref, side_ref, c2_ref, out_ref):
  num = acc_ref[0, :, 0:VAL2] + acc_ref[1, :, 0:VAL2]
  den = acc_ref[0, :, VAL2:VAL2 + 1] + acc_ref[1, :, VAL2:VAL2 + 1]
  z = num / (den + 1e-16) + c2_ref[0:1, :] - LBDA1 * side_ref[...]
  m = jnp.max(z, axis=1, keepdims=True)
  zs = z - m
  lse = jnp.log(jnp.sum(jnp.exp(zs), axis=1, keepdims=True))
  out_ref[...] = zs - lse


def kernel(x, edge_index, W1, att_src1, att_dst1, b1, W2, att_src2, att_dst2,
           b2, fc1_w, fc1_b, fc2_w, fc2_b):
  f32 = jnp.float32
  xp = jnp.pad(x, ((0, NPAD - NN), (0, 0)))
  eye8 = jnp.eye(8, dtype=f32)
  bds = (eye8[:, None, :] * att_src1.reshape(HEADS, NHID)[:, :, None]).reshape(HEADS * NHID, HEADS)
  bdd = (eye8[:, None, :] * att_dst1.reshape(HEADS, NHID)[:, :, None]).reshape(HEADS * NHID, HEADS)
  rep8 = jnp.repeat(eye8, NHID, axis=1)
  fc1t = jnp.tile(fc1_w.T, (1, HEADS))
  cvec = jnp.broadcast_to(b1 - LBDA0 * jnp.tile(fc1_b, HEADS), (8, HEADS * NHID))
  att2 = jnp.concatenate([att_src2.reshape(1, NCLASS), att_dst2.reshape(1, NCLASS),
                          jnp.zeros((6, NCLASS), f32)], axis=0)
  fc2t = fc2_w.T
  c2vec = jnp.broadcast_to(b2 - LBDA1 * fc2_b, (8, NCLASS))

  ar = jnp.arange(NN, dtype=jnp.int32)
  padidx = NN + (jnp.arange(EP - ETOT, dtype=jnp.int32) % 64)
  src = jnp.concatenate([edge_index[0], ar, padidx])
  dst = jnp.concatenate([edge_index[1], ar, padidx])

  tab_a, tab_b = pl.pallas_call(
      _tc1_body,
      grid=(NPAD // 256,),
      in_specs=[
          pl.BlockSpec((256, NFEAT), lambda i: (i, 0)),
          pl.BlockSpec((NFEAT, HEADS * NHID), lambda i: (0, 0)),
          pl.BlockSpec((HEADS * NHID, HEADS), lambda i: (0, 0)),
          pl.BlockSpec((HEADS * NHID, HEADS), lambda i: (0, 0)),
      ],
      out_specs=[
          pl.BlockSpec((256, AROW1), lambda i: (i, 0)),
          pl.BlockSpec((256, BROW1), lambda i: (i, 0)),
      ],
      out_shape=[
          jax.ShapeDtypeStruct((NPAD, AROW1), f32),
          jax.ShapeDtypeStruct((NPAD, BROW1), f32),
      ],
  )(xp, W1, bds, bdd)

  acc1 = _make_sc_edge(AROW1, ROW1, BROW1, VAL1, HEADS, NHID,
                       stage_a=False)(src, dst, tab_a, tab_b)

  taba2, tabb2, side2 = pl.pallas_call(
      _tc2_body,
      grid=(NPAD // 256,),
      in_specs=[
          pl.BlockSpec((2, 256, ROW1), lambda i: (0, i, 0)),
          pl.BlockSpec((256, NFEAT), lambda i: (i, 0)),
          pl.BlockSpec((NFEAT, HEADS * NHID), lambda i: (0, 0)),
          pl.BlockSpec((8, HEADS * NHID), lambda i: (0, 0)),
          pl.BlockSpec((HEADS, HEADS * NHID), lambda i: (0, 0)),
          pl.BlockSpec((HEADS * NHID, NCLASS), lambda i: (0, 0)),
          pl.BlockSpec((8, NCLASS), lambda i: (0, 0)),
          pl.BlockSpec((HEADS * NHID, NCLASS), lambda i: (0, 0)),
      ],
      out_specs=[
          pl.BlockSpec((256, AROW1), lambda i: (i, 0)),
          pl.BlockSpec((256, BROW2), lambda i: (i, 0)),
          pl.BlockSpec((256, NCLASS), lambda i: (i, 0)),
      ],
      out_shape=[
          jax.ShapeDtypeStruct((NPAD, AROW1), f32),
          jax.ShapeDtypeStruct((NPAD, BROW2), f32),
          jax.ShapeDtypeStruct((NPAD, NCLASS), f32),
      ],
  )(acc1, xp, fc1t, cvec, rep8, W2, att2, fc2t)

  acc2 = _make_sc_edge(AROW1, ROW2, BROW2, VAL2, 1, NCLASS,
                       stage_a=False)(src, dst, taba2, tabb2)

  out = pl.pallas_call(
      _tc3_body,
      grid=(NN // 400,),
      in_specs=[
          pl.BlockSpec((2, 400, ROW2), lambda i: (0, i, 0)),
          pl.BlockSpec((400, NCLASS), lambda i: (i, 0)),
          pl.BlockSpec((8, NCLASS), lambda i: (0, 0)),
      ],
      out_specs=pl.BlockSpec((400, NCLASS), lambda i: (i, 0)),
      out_shape=jax.ShapeDtypeStruct((NN, NCLASS), f32),
  )(acc2, side2, c2vec)

  return out
